# baseline (speedup 1.0000x reference)
import functools

import jax
import jax.numpy as jnp
from jax import lax
from jax.experimental import pallas as pl
from jax.experimental.pallas import tpu as pltpu
from jax.experimental.pallas import tpu_sc as plsc

BATCH = 16384
D_IN = 128
FEAT_DIM = 64
PAIR = 2 * FEAT_DIM

_NC = 2
_NS = 16
_NW = _NC * _NS
_BPW = BATCH // _NW


def _sc_gather_body(idx_hbm, pairs_hbm, emb_hbm, idx_v, pair_v, rows_v, sem):
    wid = lax.axis_index("s") * _NC + lax.axis_index("c")
    base = wid * _BPW
    pltpu.sync_copy(idx_hbm.at[pl.ds(base, _BPW)], idx_v)

    def halve(t, carry):
        pair_v[pl.ds(t * 16, 16)] = idx_v[pl.ds(t * 16, 16)] >> 1
        return carry

    lax.fori_loop(0, _BPW // 16, halve, 0)
    pltpu.async_copy(pairs_hbm.at[pair_v], rows_v, sem).wait()
    pltpu.sync_copy(rows_v, emb_hbm.at[pl.ds(base, _BPW)])


@functools.partial(
    pl.kernel,
    out_type=jax.ShapeDtypeStruct((BATCH, PAIR), jnp.float32),
    mesh=plsc.VectorSubcoreMesh(core_axis_name="c", subcore_axis_name="s"),
    scratch_types=[
        pltpu.VMEM((_BPW,), jnp.int32),
        pltpu.VMEM((_BPW,), jnp.int32),
        pltpu.VMEM((_BPW, PAIR), jnp.float32),
        pltpu.SemaphoreType.DMA,
    ],
)
def _sc_gather(idx_hbm, pairs_hbm, emb_hbm, idx_v, pair_v, rows_v, sem):
    _sc_gather_body(idx_hbm, pairs_hbm, emb_hbm, idx_v, pair_v, rows_v, sem)


def _tc_body(feat_ref, w_ref, b_ref, emb_ref, idx_ref, out_ref):
    proj = (
        lax.dot_general(
            feat_ref[...], w_ref[...],
            (((1,), (1,)), ((), ())),
            preferred_element_type=jnp.float32,
        )
        + b_ref[...]
    )
    pair_rows = emb_ref[...]
    odd = (idx_ref[...] & 1) == 1
    sel = jnp.where(odd, pair_rows[:, FEAT_DIM:], pair_rows[:, :FEAT_DIM])
    out_ref[...] = jnp.concatenate([proj, sel], axis=-1)


def _tc_project_concat(feat, W, b, emb, idx):
    blk = 2048
    grid = BATCH // blk
    return pl.pallas_call(
        _tc_body,
        grid=(grid,),
        in_specs=[
            pl.BlockSpec((blk, D_IN), lambda i: (i, 0)),
            pl.BlockSpec((FEAT_DIM, D_IN), lambda i: (0, 0)),
            pl.BlockSpec((1, FEAT_DIM), lambda i: (0, 0)),
            pl.BlockSpec((blk, PAIR), lambda i: (i, 0)),
            pl.BlockSpec((blk, 1), lambda i: (i, 0)),
        ],
        out_specs=pl.BlockSpec((blk, D_IN), lambda i: (i, 0)),
        out_shape=jax.ShapeDtypeStruct((BATCH, D_IN), jnp.float32),
    )(feat, W, b.reshape(1, FEAT_DIM), emb, idx.reshape(BATCH, 1))


def kernel(feat, idx, W, b, table):
    idx32 = idx.astype(jnp.int32)
    pairs = jnp.reshape(table, (table.shape[0] // 2, PAIR))
    emb = _sc_gather(idx32, pairs)
    return _tc_project_concat(feat, W, b, emb, idx32)


# speedup vs baseline: 1.2879x; 1.2879x over previous
import functools

import jax
import jax.numpy as jnp
from jax import lax
from jax.experimental import pallas as pl
from jax.experimental.pallas import tpu as pltpu
from jax.experimental.pallas import tpu_sc as plsc

BATCH = 16384
D_IN = 128
FEAT_DIM = 64
PAIR = 2 * FEAT_DIM

_NC = 2
_NS = 16
_NW = _NC * _NS
_BPW = BATCH // _NW


def _sc_gather_body(idx_hbm, pairs_hbm, emb_hbm, idx_v, pair_v, rows_v, sem):
    wid = lax.axis_index("s") * _NC + lax.axis_index("c")
    base = wid * _BPW
    pltpu.sync_copy(idx_hbm.at[pl.ds(base, _BPW)], idx_v)

    def halve(t, carry):
        r = idx_v[pl.ds(t * 16, 16)]
        pair_v[pl.ds(t * 16, 16)] = ((r >> 7) << 6) + (r & 63)
        return carry

    lax.fori_loop(0, _BPW // 16, halve, 0)
    pltpu.async_copy(pairs_hbm.at[pair_v], rows_v, sem).wait()
    pltpu.sync_copy(rows_v, emb_hbm.at[pl.ds(base, _BPW)])


@functools.partial(
    pl.kernel,
    out_type=jax.ShapeDtypeStruct((BATCH, PAIR), jnp.float32),
    mesh=plsc.VectorSubcoreMesh(core_axis_name="c", subcore_axis_name="s"),
    scratch_types=[
        pltpu.VMEM((_BPW,), jnp.int32),
        pltpu.VMEM((_BPW,), jnp.int32),
        pltpu.VMEM((_BPW, PAIR), jnp.float32),
        pltpu.SemaphoreType.DMA,
    ],
)
def _sc_gather(idx_hbm, pairs_hbm, emb_hbm, idx_v, pair_v, rows_v, sem):
    _sc_gather_body(idx_hbm, pairs_hbm, emb_hbm, idx_v, pair_v, rows_v, sem)


def _tc_body(feat_ref, w_ref, b_ref, emb_ref, idx_ref, out_ref):
    proj = (
        lax.dot_general(
            feat_ref[...], w_ref[...],
            (((1,), (1,)), ((), ())),
            preferred_element_type=jnp.float32,
        )
        + b_ref[...]
    )
    pair_rows = emb_ref[...]
    odd = ((idx_ref[...] >> 6) & 1) == 1
    sel = jnp.where(odd, pair_rows[:, FEAT_DIM:], pair_rows[:, :FEAT_DIM])
    out_ref[...] = jnp.concatenate([proj, sel], axis=-1)


def _tc_project_concat(feat, W, b, emb, idx):
    blk = 2048
    grid = BATCH // blk
    return pl.pallas_call(
        _tc_body,
        grid=(grid,),
        in_specs=[
            pl.BlockSpec((blk, D_IN), lambda i: (i, 0)),
            pl.BlockSpec((FEAT_DIM, D_IN), lambda i: (0, 0)),
            pl.BlockSpec((1, FEAT_DIM), lambda i: (0, 0)),
            pl.BlockSpec((blk, PAIR), lambda i: (i, 0)),
            pl.BlockSpec((blk, 1), lambda i: (i, 0)),
        ],
        out_specs=pl.BlockSpec((blk, D_IN), lambda i: (i, 0)),
        out_shape=jax.ShapeDtypeStruct((BATCH, D_IN), jnp.float32),
    )(feat, W, b.reshape(1, FEAT_DIM), emb, idx.reshape(BATCH, 1))


_VOCAB = 1000000
_TBLK = 2048
_NPAIR = 500032


def _tc_transpose_body(xt_ref, out_ref):
    ident = jnp.eye(FEAT_DIM, dtype=jnp.float32)
    t = lax.dot_general(
        xt_ref[...], ident, (((0,), (0,)), ((), ())),
        preferred_element_type=jnp.float32,
    )
    tl = jnp.concatenate(
        [t[k * 128:k * 128 + 64] for k in range(_TBLK // 128)], axis=0)
    tr = jnp.concatenate(
        [t[k * 128 + 64:(k + 1) * 128] for k in range(_TBLK // 128)], axis=0)
    out_ref[...] = jnp.concatenate([tl, tr], axis=-1)


def _tc_pairs(table):
    grid = (_VOCAB + _TBLK - 1) // _TBLK
    return pl.pallas_call(
        _tc_transpose_body,
        grid=(grid,),
        in_specs=[pl.BlockSpec((FEAT_DIM, _TBLK), lambda i: (0, i))],
        out_specs=pl.BlockSpec((_TBLK // 2, PAIR), lambda i: (i, 0)),
        out_shape=jax.ShapeDtypeStruct((_NPAIR, PAIR), jnp.float32),
    )(table.T)


def kernel(feat, idx, W, b, table):
    idx32 = idx.astype(jnp.int32)
    pairs = _tc_pairs(table)
    emb = _sc_gather(idx32, pairs)
    return _tc_project_concat(feat, W, b, emb, idx32)


# speedup vs baseline: 2.1508x; 1.6700x over previous
import functools

import jax
import jax.numpy as jnp
from jax import lax
from jax.experimental import pallas as pl
from jax.experimental.pallas import tpu as pltpu
from jax.experimental.pallas import tpu_sc as plsc

BATCH = 16384
D_IN = 128
FEAT_DIM = 64
PAIR = 2 * FEAT_DIM

_NC = 2
_NS = 16
_NW = _NC * _NS
_BPW = BATCH // _NW


def _sc_gather_body(idx_hbm, pairs_hbm, emb_hbm, idx_v, pair_v, rows_v, sem):
    wid = lax.axis_index("s") * _NC + lax.axis_index("c")
    base = wid * _BPW
    pltpu.sync_copy(idx_hbm.at[pl.ds(base, _BPW)], idx_v)

    def halve(t, carry):
        r = idx_v[pl.ds(t * 16, 16)]
        pair_v[pl.ds(t * 16, 16)] = ((r >> 7) << 6) + (r & 63)
        return carry

    lax.fori_loop(0, _BPW // 16, halve, 0)
    pltpu.async_copy(pairs_hbm.at[pair_v], rows_v, sem).wait()
    pltpu.sync_copy(rows_v, emb_hbm.at[pl.ds(base, _BPW)])


@functools.partial(
    pl.kernel,
    out_type=jax.ShapeDtypeStruct((BATCH, PAIR), jnp.float32),
    mesh=plsc.VectorSubcoreMesh(core_axis_name="c", subcore_axis_name="s"),
    scratch_types=[
        pltpu.VMEM((_BPW,), jnp.int32),
        pltpu.VMEM((_BPW,), jnp.int32),
        pltpu.VMEM((_BPW, PAIR), jnp.float32),
        pltpu.SemaphoreType.DMA,
    ],
)
def _sc_gather(idx_hbm, pairs_hbm, emb_hbm, idx_v, pair_v, rows_v, sem):
    _sc_gather_body(idx_hbm, pairs_hbm, emb_hbm, idx_v, pair_v, rows_v, sem)


def _tc_body(feat_ref, w_ref, b_ref, emb_ref, idx_ref, out_ref):
    proj = (
        lax.dot_general(
            feat_ref[...], w_ref[...],
            (((1,), (1,)), ((), ())),
            preferred_element_type=jnp.float32,
        )
        + b_ref[...]
    )
    pair_rows = emb_ref[...]
    odd = ((idx_ref[...] >> 6) & 1) == 1
    sel = jnp.where(odd, pair_rows[:, FEAT_DIM:], pair_rows[:, :FEAT_DIM])
    out_ref[...] = jnp.concatenate([proj, sel], axis=-1)


def _tc_project_concat(feat, W, b, emb, idx):
    blk = 2048
    grid = BATCH // blk
    return pl.pallas_call(
        _tc_body,
        grid=(grid,),
        in_specs=[
            pl.BlockSpec((blk, D_IN), lambda i: (i, 0)),
            pl.BlockSpec((FEAT_DIM, D_IN), lambda i: (0, 0)),
            pl.BlockSpec((1, FEAT_DIM), lambda i: (0, 0)),
            pl.BlockSpec((blk, PAIR), lambda i: (i, 0)),
            pl.BlockSpec((blk, 1), lambda i: (i, 0)),
        ],
        out_specs=pl.BlockSpec((blk, D_IN), lambda i: (i, 0)),
        out_shape=jax.ShapeDtypeStruct((BATCH, D_IN), jnp.float32),
    )(feat, W, b.reshape(1, FEAT_DIM), emb, idx.reshape(BATCH, 1))


_VOCAB = 1000000
_TBLK = 8192
_NPAIR = 500032


def _tc_transpose_body(xt_ref, out_ref):
    ident = jnp.eye(FEAT_DIM, dtype=jnp.float32)
    t = lax.dot_general(
        xt_ref[...], ident, (((0,), (0,)), ((), ())),
        preferred_element_type=jnp.float32,
    )
    tl = jnp.concatenate(
        [t[k * 128:k * 128 + 64] for k in range(_TBLK // 128)], axis=0)
    tr = jnp.concatenate(
        [t[k * 128 + 64:(k + 1) * 128] for k in range(_TBLK // 128)], axis=0)
    out_ref[...] = jnp.concatenate([tl, tr], axis=-1)


def _tc_pairs(table):
    grid = (_VOCAB + _TBLK - 1) // _TBLK
    return pl.pallas_call(
        _tc_transpose_body,
        grid=(grid,),
        in_specs=[pl.BlockSpec((FEAT_DIM, _TBLK), lambda i: (0, i))],
        out_specs=pl.BlockSpec((_TBLK // 2, PAIR), lambda i: (i, 0)),
        out_shape=jax.ShapeDtypeStruct((_NPAIR, PAIR), jnp.float32),
    )(table.T)


def kernel(feat, idx, W, b, table):
    idx32 = idx.astype(jnp.int32)
    pairs = _tc_pairs(table)
    emb = _sc_gather(idx32, pairs)
    return _tc_project_concat(feat, W, b, emb, idx32)


# speedup vs baseline: 2.4324x; 1.1309x over previous
import functools

import jax
import jax.numpy as jnp
from jax import lax
from jax.experimental import pallas as pl
from jax.experimental.pallas import tpu as pltpu
from jax.experimental.pallas import tpu_sc as plsc

BATCH = 16384
D_IN = 128
FEAT_DIM = 64
PAIR = 2 * FEAT_DIM

_NC = 2
_NS = 16
_NW = _NC * _NS
_BPW = BATCH // _NW


def _sc_gather_body(idx_hbm, pairs_hbm, emb_hbm, idx_v, pair_v, rows_v, sem):
    wid = lax.axis_index("s") * _NC + lax.axis_index("c")
    base = wid * _BPW
    pltpu.sync_copy(idx_hbm.at[pl.ds(base, _BPW)], idx_v)

    def halve(t, carry):
        r = idx_v[pl.ds(t * 16, 16)]
        pair_v[pl.ds(t * 16, 16)] = ((r >> 7) << 6) + (r & 63)
        return carry

    lax.fori_loop(0, _BPW // 16, halve, 0)
    pltpu.async_copy(pairs_hbm.at[pair_v], rows_v, sem).wait()
    pltpu.sync_copy(rows_v, emb_hbm.at[pl.ds(base, _BPW)])


@functools.partial(
    pl.kernel,
    out_type=jax.ShapeDtypeStruct((BATCH, PAIR), jnp.float32),
    mesh=plsc.VectorSubcoreMesh(core_axis_name="c", subcore_axis_name="s"),
    scratch_types=[
        pltpu.VMEM((_BPW,), jnp.int32),
        pltpu.VMEM((_BPW,), jnp.int32),
        pltpu.VMEM((_BPW, PAIR), jnp.float32),
        pltpu.SemaphoreType.DMA,
    ],
)
def _sc_gather(idx_hbm, pairs_hbm, emb_hbm, idx_v, pair_v, rows_v, sem):
    _sc_gather_body(idx_hbm, pairs_hbm, emb_hbm, idx_v, pair_v, rows_v, sem)


def _tc_body(feat_ref, w_ref, b_ref, emb_ref, idx_ref, out_ref):
    proj = (
        lax.dot_general(
            feat_ref[...], w_ref[...],
            (((1,), (1,)), ((), ())),
            preferred_element_type=jnp.float32,
        )
        + b_ref[...]
    )
    pair_rows = emb_ref[...]
    odd = ((idx_ref[...] >> 6) & 1) == 1
    sel = jnp.where(odd, pair_rows[:, FEAT_DIM:], pair_rows[:, :FEAT_DIM])
    out_ref[...] = jnp.concatenate([proj, sel], axis=-1)


def _tc_project_concat(feat, W, b, emb, idx):
    blk = 2048
    grid = BATCH // blk
    return pl.pallas_call(
        _tc_body,
        grid=(grid,),
        in_specs=[
            pl.BlockSpec((blk, D_IN), lambda i: (i, 0)),
            pl.BlockSpec((FEAT_DIM, D_IN), lambda i: (0, 0)),
            pl.BlockSpec((1, FEAT_DIM), lambda i: (0, 0)),
            pl.BlockSpec((blk, PAIR), lambda i: (i, 0)),
            pl.BlockSpec((blk, 1), lambda i: (i, 0)),
        ],
        out_specs=pl.BlockSpec((blk, D_IN), lambda i: (i, 0)),
        out_shape=jax.ShapeDtypeStruct((BATCH, D_IN), jnp.float32),
    )(feat, W, b.reshape(1, FEAT_DIM), emb, idx.reshape(BATCH, 1))


_VOCAB = 1000000
_TBLK = 16384
_NPAIR = 500032


def _tc_transpose_body(xt_ref, out_ref):
    ident = jnp.eye(FEAT_DIM, dtype=jnp.float32)
    t = lax.dot_general(
        xt_ref[...], ident, (((0,), (0,)), ((), ())),
        preferred_element_type=jnp.float32,
    )
    tl = jnp.concatenate(
        [t[k * 128:k * 128 + 64] for k in range(_TBLK // 128)], axis=0)
    tr = jnp.concatenate(
        [t[k * 128 + 64:(k + 1) * 128] for k in range(_TBLK // 128)], axis=0)
    out_ref[...] = jnp.concatenate([tl, tr], axis=-1)


def _tc_pairs(table):
    grid = (_VOCAB + _TBLK - 1) // _TBLK
    return pl.pallas_call(
        _tc_transpose_body,
        grid=(grid,),
        in_specs=[pl.BlockSpec((FEAT_DIM, _TBLK), lambda i: (0, i))],
        out_specs=pl.BlockSpec((_TBLK // 2, PAIR), lambda i: (i, 0)),
        out_shape=jax.ShapeDtypeStruct((_NPAIR, PAIR), jnp.float32),
    )(table.T)


def kernel(feat, idx, W, b, table):
    idx32 = idx.astype(jnp.int32)
    pairs = _tc_pairs(table)
    emb = _sc_gather(idx32, pairs)
    return _tc_project_concat(feat, W, b, emb, idx32)


# speedup vs baseline: 2.5637x; 1.0540x over previous
import functools

import jax
import jax.numpy as jnp
from jax import lax
from jax.experimental import pallas as pl
from jax.experimental.pallas import tpu as pltpu
from jax.experimental.pallas import tpu_sc as plsc

BATCH = 16384
D_IN = 128
FEAT_DIM = 64
PAIR = 2 * FEAT_DIM

_NC = 2
_NS = 16
_NW = _NC * _NS
_BPW = BATCH // _NW


def _sc_gather_body(idx_hbm, pairs_hbm, emb_hbm, idx_v, pair_v, rows_v, sem):
    wid = lax.axis_index("s") * _NC + lax.axis_index("c")
    base = wid * _BPW
    pltpu.sync_copy(idx_hbm.at[pl.ds(base, _BPW)], idx_v)

    def halve(t, carry):
        r = idx_v[pl.ds(t * 16, 16)]
        pair_v[pl.ds(t * 16, 16)] = ((r >> 7) << 6) + (r & 63)
        return carry

    lax.fori_loop(0, _BPW // 16, halve, 0)
    pltpu.async_copy(pairs_hbm.at[pair_v], rows_v, sem).wait()
    pltpu.sync_copy(rows_v, emb_hbm.at[pl.ds(base, _BPW)])


@functools.partial(
    pl.kernel,
    out_type=jax.ShapeDtypeStruct((BATCH, PAIR), jnp.float32),
    mesh=plsc.VectorSubcoreMesh(core_axis_name="c", subcore_axis_name="s"),
    scratch_types=[
        pltpu.VMEM((_BPW,), jnp.int32),
        pltpu.VMEM((_BPW,), jnp.int32),
        pltpu.VMEM((_BPW, PAIR), jnp.float32),
        pltpu.SemaphoreType.DMA,
    ],
)
def _sc_gather(idx_hbm, pairs_hbm, emb_hbm, idx_v, pair_v, rows_v, sem):
    _sc_gather_body(idx_hbm, pairs_hbm, emb_hbm, idx_v, pair_v, rows_v, sem)


def _tc_body(feat_ref, w_ref, b_ref, emb_ref, idx_ref, out_ref):
    proj = (
        lax.dot_general(
            feat_ref[...], w_ref[...],
            (((1,), (1,)), ((), ())),
            preferred_element_type=jnp.float32,
        )
        + b_ref[...]
    )
    pair_rows = emb_ref[...]
    odd = ((idx_ref[...] >> 6) & 1) == 1
    sel = jnp.where(odd, pair_rows[:, FEAT_DIM:], pair_rows[:, :FEAT_DIM])
    out_ref[...] = jnp.concatenate([proj, sel], axis=-1)


def _tc_project_concat(feat, W, b, emb, idx):
    blk = 2048
    grid = BATCH // blk
    return pl.pallas_call(
        _tc_body,
        grid=(grid,),
        in_specs=[
            pl.BlockSpec((blk, D_IN), lambda i: (i, 0)),
            pl.BlockSpec((FEAT_DIM, D_IN), lambda i: (0, 0)),
            pl.BlockSpec((1, FEAT_DIM), lambda i: (0, 0)),
            pl.BlockSpec((blk, PAIR), lambda i: (i, 0)),
            pl.BlockSpec((blk, 1), lambda i: (i, 0)),
        ],
        out_specs=pl.BlockSpec((blk, D_IN), lambda i: (i, 0)),
        out_shape=jax.ShapeDtypeStruct((BATCH, D_IN), jnp.float32),
    )(feat, W, b.reshape(1, FEAT_DIM), emb, idx.reshape(BATCH, 1))


_VOCAB = 1000000
_TBLK = 32768
_NPAIR = 500032


def _tc_transpose_body(xt_ref, out_ref):
    ident = jnp.eye(FEAT_DIM, dtype=jnp.float32)
    t = lax.dot_general(
        xt_ref[...], ident, (((0,), (0,)), ((), ())),
        preferred_element_type=jnp.float32,
    )
    tl = jnp.concatenate(
        [t[k * 128:k * 128 + 64] for k in range(_TBLK // 128)], axis=0)
    tr = jnp.concatenate(
        [t[k * 128 + 64:(k + 1) * 128] for k in range(_TBLK // 128)], axis=0)
    out_ref[...] = jnp.concatenate([tl, tr], axis=-1)


def _tc_pairs(table):
    grid = (_VOCAB + _TBLK - 1) // _TBLK
    return pl.pallas_call(
        _tc_transpose_body,
        grid=(grid,),
        in_specs=[pl.BlockSpec((FEAT_DIM, _TBLK), lambda i: (0, i))],
        out_specs=pl.BlockSpec((_TBLK // 2, PAIR), lambda i: (i, 0)),
        out_shape=jax.ShapeDtypeStruct((_NPAIR, PAIR), jnp.float32),
    )(table.T)


def kernel(feat, idx, W, b, table):
    idx32 = idx.astype(jnp.int32)
    pairs = _tc_pairs(table)
    emb = _sc_gather(idx32, pairs)
    return _tc_project_concat(feat, W, b, emb, idx32)
